# baseline (speedup 1.0000x reference)
import functools

import jax
import jax.numpy as jnp
from jax import lax
from jax.experimental import pallas as pl
from jax.experimental.pallas import tpu as pltpu
from jax.experimental.pallas import tpu_sc as plsc

L = 16
NC, NS = 2, 16
NW = NC * NS
GR = 128
GB = 104


def _fm_call(x_t, emb_cols, fc_flat, bias16, B, F, D, total):
    field_size = total // F
    rpw = B // NW
    nch = rpw // GR
    ppc = GR * F
    ng = ppc // L
    ngath = ppc // GB

    mesh = plsc.VectorSubcoreMesh(
        core_axis_name="c", subcore_axis_name="s", num_cores=NC, num_subcores=NS)

    @functools.partial(
        pl.kernel,
        out_type=jax.ShapeDtypeStruct((B,), jnp.float32),
        mesh=mesh,
        scratch_types=[
            pltpu.VMEM((ppc,), jnp.int32),
            pltpu.VMEM((ppc,), jnp.int32),
            *[pltpu.VMEM((ppc,), jnp.float32) for _ in range(D)],
            pltpu.VMEM((ppc,), jnp.float32),
            pltpu.VMEM((rpw,), jnp.float32),
            pltpu.VMEM((L,), jnp.float32),
            pltpu.SemaphoreType.DMA,
        ],
        compiler_params=pltpu.CompilerParams(
            needs_layout_passes=False, use_tc_tiling_on_sc=False),
    )
    def fm(*args):
        cols_hbm = args[:D]
        fc_hbm, x_hbm, b_hbm, out_hbm, xbuf, idxbuf = args[D:D + 6]
        ebufs = args[D + 6:2 * D + 6]
        fcbuf, obuf, bbuf, sem = args[2 * D + 6:]
        w = lax.axis_index("s") * NC + lax.axis_index("c")
        pltpu.sync_copy(b_hbm, bbuf)
        iota = lax.iota(jnp.int32, L)
        m_last = iota == (L - 1)
        zero_i = iota * 0
        bias_v = bbuf[pl.ds(0, L)]

        def chunk_body(c, _):
            pltpu.sync_copy(x_hbm.at[pl.ds((w * nch + c) * ppc, ppc)], xbuf)

            def idx_body(g, _):
                off = pl.multiple_of(g * L, L)
                xv = xbuf[pl.ds(off, L)]
                idxbuf[pl.ds(off, L)] = xv + (g >> 3) * field_size
                return 0

            lax.fori_loop(0, ng, idx_body, 0)

            copies = [pltpu.async_copy(fc_hbm.at[idxbuf], fcbuf, sem)]
            for d in range(D):
                copies.append(pltpu.async_copy(
                    cols_hbm[d].at[idxbuf], ebufs[d], sem))
            for cp in copies:
                cp.wait()

            def rows_body(i, _):
                rr = i * L
                lin = bias_v
                for f in range(F):
                    lin = lin + fcbuf[pl.ds(f * GR + rr, L)]
                zacc = lin
                for d in range(D):
                    s = None
                    ss = None
                    for f in range(F):
                        e = ebufs[d][pl.ds(f * GR + rr, L)]
                        s = e if s is None else s + e
                        ss = e * e if ss is None else ss + e * e
                    zacc = zacc + 0.5 * (s * s - ss)
                obuf[pl.ds(c * GR + rr, L)] = 1.0 / (1.0 + jnp.exp(-zacc))
                return 0

            lax.fori_loop(0, GR // L, rows_body, 0)
            return 0

        lax.fori_loop(0, nch, chunk_body, 0)
        pltpu.sync_copy(obuf, out_hbm.at[pl.ds(w * rpw, rpw)])

    return fm(*emb_cols, fc_flat, x_t, bias16)


def kernel(x, emb_table, fc_table, bias):
    B, F = x.shape
    total, D = emb_table.shape
    assert D == L and B % (NW * GR) == 0 and total % F == 0
    x_t = jnp.transpose(
        x.astype(jnp.int32).reshape(B // GR, GR, F), (0, 2, 1)).reshape(-1)
    emb_cols = [lax.optimization_barrier(emb_table[:, d]) for d in range(D)]
    fc_flat = fc_table.reshape(-1)
    bias16 = jnp.broadcast_to(bias.astype(jnp.float32), (L,))
    return _fm_call(x_t, emb_cols, fc_flat, bias16, B, F, D, total)


# speedup vs baseline: 1.1140x; 1.1140x over previous
import functools

import jax
import jax.numpy as jnp
from jax import lax
from jax.experimental import pallas as pl
from jax.experimental.pallas import tpu as pltpu
from jax.experimental.pallas import tpu_sc as plsc

L = 16
NC, NS = 2, 16
NW = NC * NS
GR = 128


def _fm_stage(cols, x_t, fc_flat, bias16, zprev, B, F, total, final):
    nd = len(cols)
    field_size = total // F
    rpw = B // NW
    nch = rpw // GR
    ppc = GR * F
    ng = ppc // L

    mesh = plsc.VectorSubcoreMesh(
        core_axis_name="c", subcore_axis_name="s", num_cores=NC, num_subcores=NS)

    @functools.partial(
        pl.kernel,
        out_type=jax.ShapeDtypeStruct((B,), jnp.float32),
        mesh=mesh,
        scratch_types=[
            pltpu.VMEM((ppc,), jnp.int32),
            pltpu.VMEM((ppc,), jnp.int32),
            *[pltpu.VMEM((ppc,), jnp.float32) for _ in range(nd)],
            pltpu.VMEM((ppc,), jnp.float32),
            pltpu.VMEM((rpw,), jnp.float32),
            pltpu.VMEM((L,), jnp.float32),
            pltpu.SemaphoreType.DMA,
        ],
        compiler_params=pltpu.CompilerParams(
            needs_layout_passes=False, use_tc_tiling_on_sc=False),
    )
    def fm(*args):
        cols_hbm = args[:nd]
        aux_hbm, x_hbm, b_hbm, out_hbm, xbuf, idxbuf = args[nd:nd + 6]
        ebufs = args[nd + 6:2 * nd + 6]
        fcbuf, obuf, bbuf, sem = args[2 * nd + 6:]
        w = lax.axis_index("s") * NC + lax.axis_index("c")
        pltpu.sync_copy(b_hbm, bbuf)
        bias_v = bbuf[pl.ds(0, L)]

        def chunk_body(c, _):
            pltpu.sync_copy(x_hbm.at[pl.ds((w * nch + c) * ppc, ppc)], xbuf)

            def idx_body(g, _):
                off = pl.multiple_of(g * L, L)
                xv = xbuf[pl.ds(off, L)]
                idxbuf[pl.ds(off, L)] = xv + (g >> 3) * field_size
                return 0

            lax.fori_loop(0, ng, idx_body, 0)

            copies = []
            if not final:
                copies.append(pltpu.async_copy(aux_hbm.at[idxbuf], fcbuf, sem))
            else:
                pltpu.sync_copy(
                    aux_hbm.at[pl.ds((w * nch + c) * GR, GR)],
                    fcbuf.at[pl.ds(0, GR)])
            for d in range(nd):
                copies.append(pltpu.async_copy(
                    cols_hbm[d].at[idxbuf], ebufs[d], sem))
            for cp in copies:
                cp.wait()

            def rows_body(i, _):
                rr = i * L
                if final:
                    zacc = fcbuf[pl.ds(rr, L)]
                else:
                    zacc = bias_v
                    for f in range(F):
                        zacc = zacc + fcbuf[pl.ds(f * GR + rr, L)]
                for d in range(nd):
                    s = None
                    ss = None
                    for f in range(F):
                        e = ebufs[d][pl.ds(f * GR + rr, L)]
                        s = e if s is None else s + e
                        ss = e * e if ss is None else ss + e * e
                    zacc = zacc + 0.5 * (s * s - ss)
                if final:
                    zacc = 1.0 / (1.0 + jnp.exp(-zacc))
                obuf[pl.ds(c * GR + rr, L)] = zacc
                return 0

            lax.fori_loop(0, GR // L, rows_body, 0)
            return 0

        lax.fori_loop(0, nch, chunk_body, 0)
        pltpu.sync_copy(obuf, out_hbm.at[pl.ds(w * rpw, rpw)])

    aux = zprev if final else fc_flat
    return fm(*cols, aux, x_t, bias16)


def kernel(x, emb_table, fc_table, bias):
    B, F = x.shape
    total, D = emb_table.shape
    assert D == L and B % (NW * GR) == 0 and total % F == 0
    x_t = jnp.transpose(
        x.astype(jnp.int32).reshape(B // GR, GR, F), (0, 2, 1)).reshape(-1)
    emb_cols = [emb_table[:, d] for d in range(D)]
    fc_flat = fc_table.reshape(-1)
    bias16 = jnp.broadcast_to(bias.astype(jnp.float32), (L,))
    za = _fm_stage(emb_cols[:D // 2], x_t, fc_flat, bias16, None,
                   B, F, total, final=False)
    return _fm_stage(emb_cols[D // 2:], x_t, fc_flat, bias16, za,
                     B, F, total, final=True)


# speedup vs baseline: 1.1827x; 1.0616x over previous
import functools

import jax
import jax.numpy as jnp
from jax import lax
from jax.experimental import pallas as pl
from jax.experimental.pallas import tpu as pltpu
from jax.experimental.pallas import tpu_sc as plsc

L = 16
NC, NS = 2, 16
NW = NC * NS
GR = 128


def _fm_stage(cols, x_t, fc_flat, bias16, zprev, B, F, total, first, final):
    nd = len(cols)
    field_size = total // F
    rpw = B // NW
    nch = rpw // GR
    ppc = GR * F
    ng = ppc // L

    mesh = plsc.VectorSubcoreMesh(
        core_axis_name="c", subcore_axis_name="s", num_cores=NC, num_subcores=NS)

    @functools.partial(
        pl.kernel,
        out_type=jax.ShapeDtypeStruct((B,), jnp.float32),
        mesh=mesh,
        scratch_types=[
            pltpu.VMEM((ppc,), jnp.int32),
            pltpu.VMEM((ppc,), jnp.int32),
            *[pltpu.VMEM((ppc,), jnp.float32) for _ in range(nd)],
            pltpu.VMEM((ppc,), jnp.float32),
            pltpu.VMEM((rpw,), jnp.float32),
            pltpu.VMEM((L,), jnp.float32),
            pltpu.SemaphoreType.DMA,
        ],
        compiler_params=pltpu.CompilerParams(
            needs_layout_passes=False, use_tc_tiling_on_sc=False),
    )
    def fm(*args):
        cols_hbm = args[:nd]
        aux_hbm, x_hbm, b_hbm, out_hbm, xbuf, idxbuf = args[nd:nd + 6]
        ebufs = args[nd + 6:2 * nd + 6]
        fcbuf, obuf, bbuf, sem = args[2 * nd + 6:]
        w = lax.axis_index("s") * NC + lax.axis_index("c")
        pltpu.sync_copy(b_hbm, bbuf)
        bias_v = bbuf[pl.ds(0, L)]

        def chunk_body(c, _):
            pltpu.sync_copy(x_hbm.at[pl.ds((w * nch + c) * ppc, ppc)], xbuf)

            def idx_body(g, _):
                off = pl.multiple_of(g * L, L)
                xv = xbuf[pl.ds(off, L)]
                idxbuf[pl.ds(off, L)] = xv + (g >> 3) * field_size
                return 0

            lax.fori_loop(0, ng, idx_body, 0)

            copies = []
            if first:
                copies.append(pltpu.async_copy(aux_hbm.at[idxbuf], fcbuf, sem))
            else:
                pltpu.sync_copy(
                    aux_hbm.at[pl.ds((w * nch + c) * GR, GR)],
                    fcbuf.at[pl.ds(0, GR)])
            for d in range(nd):
                copies.append(pltpu.async_copy(
                    cols_hbm[d].at[idxbuf], ebufs[d], sem))
            for cp in copies:
                cp.wait()

            def rows_body(i, _):
                rr = i * L
                if first:
                    zacc = bias_v
                    for f in range(F):
                        zacc = zacc + fcbuf[pl.ds(f * GR + rr, L)]
                else:
                    zacc = fcbuf[pl.ds(rr, L)]
                for d in range(nd):
                    s = None
                    ss = None
                    for f in range(F):
                        e = ebufs[d][pl.ds(f * GR + rr, L)]
                        s = e if s is None else s + e
                        ss = e * e if ss is None else ss + e * e
                    zacc = zacc + 0.5 * (s * s - ss)
                if final:
                    zacc = 1.0 / (1.0 + jnp.exp(-zacc))
                obuf[pl.ds(c * GR + rr, L)] = zacc
                return 0

            lax.fori_loop(0, GR // L, rows_body, 0)
            return 0

        lax.fori_loop(0, nch, chunk_body, 0)
        pltpu.sync_copy(obuf, out_hbm.at[pl.ds(w * rpw, rpw)])

    aux = fc_flat if first else zprev
    return fm(*cols, aux, x_t, bias16)


def kernel(x, emb_table, fc_table, bias):
    B, F = x.shape
    total, D = emb_table.shape
    assert D == L and B % (NW * GR) == 0 and total % F == 0
    x_t = jnp.transpose(
        x.astype(jnp.int32).reshape(B // GR, GR, F), (0, 2, 1)).reshape(-1)
    emb_cols = [emb_table[:, d] for d in range(D)]
    fc_flat = fc_table.reshape(-1)
    bias16 = jnp.broadcast_to(bias.astype(jnp.float32), (L,))
    nstage = 4
    per = D // nstage
    z = None
    for k in range(nstage):
        z = _fm_stage(emb_cols[k * per:(k + 1) * per], x_t, fc_flat, bias16,
                      z, B, F, total, first=(k == 0), final=(k == nstage - 1))
    return z
